# baseline (speedup 1.0000x reference)
import functools

import jax
import jax.numpy as jnp
from jax import lax
from jax.experimental import pallas as pl
from jax.experimental.pallas import tpu as pltpu
from jax.experimental.pallas import tpu_sc as plsc

NROWS = 16384
DIM = 4096
HALF = DIM // 2
NC, NS, L = 2, 16, 16
NW = NC * NS
ROWS_PER_W = NROWS // NW
R = 4
NCHUNK = ROWS_PER_W // R
NGROUP = HALF // L
NIBUF = 2
NOBUF = 4

_mesh = plsc.VectorSubcoreMesh(
    core_axis_name="c", subcore_axis_name="s", num_cores=NC, num_subcores=NS
)


@functools.partial(
    pl.kernel,
    out_type=jax.ShapeDtypeStruct((NROWS, DIM), jnp.float32),
    mesh=_mesh,
    compiler_params=pltpu.CompilerParams(needs_layout_passes=False),
    scratch_types=[
        pltpu.VMEM((HALF,), jnp.int32),
        pltpu.VMEM((R, DIM), jnp.float32),
        pltpu.VMEM((R, DIM), jnp.float32),
        pltpu.VMEM((R, DIM), jnp.float32),
        pltpu.VMEM((R, DIM), jnp.float32),
        pltpu.VMEM((R, DIM), jnp.float32),
        pltpu.VMEM((R, DIM), jnp.float32),
        pltpu.SemaphoreType.DMA,
        pltpu.SemaphoreType.DMA,
        pltpu.SemaphoreType.DMA,
        pltpu.SemaphoreType.DMA,
        pltpu.SemaphoreType.DMA,
        pltpu.SemaphoreType.DMA,
    ],
)
def _permute_cols(x_hbm, perm_hbm, out_hbm, idx_v, in0, in1, o0, o1, o2, o3,
                  is0, is1, os0, os1, os2, os3):
    ins, outs = (in0, in1), (o0, o1, o2, o3)
    isems, osems = (is0, is1), (os0, os1, os2, os3)
    wid = lax.axis_index("s") * NC + lax.axis_index("c")
    base = wid * ROWS_PER_W
    pltpu.sync_copy(perm_hbm, idx_v)

    def in_slice(c):
        return x_hbm.at[pl.ds(base + c * R, R)]

    def out_slice(c):
        return out_hbm.at[pl.ds(base + c * R, R)]

    for b in range(NIBUF):
        pltpu.async_copy(in_slice(b), ins[b], isems[b])

    def quad_body(p, _):
        for b in range(NOBUF):
            c = NOBUF * p + b
            inb, outb = ins[b % NIBUF], outs[b]
            pltpu.make_async_copy(in_slice(c), inb, isems[b % NIBUF]).wait()

            @pl.when(p >= 1)
            def _():
                pltpu.make_async_copy(outb, out_slice(c - NOBUF), osems[b]).wait()

            @plsc.parallel_loop(0, NGROUP, 1, unroll=4)
            def _(g):
                packed = idx_v[pl.ds(g * L, L)]
                lo = jnp.bitwise_and(packed, jnp.full((L,), 0xFFFF, jnp.int32))
                hi = lax.shift_right_logical(packed, jnp.full((L,), 16, jnp.int32))
                for r in range(R):
                    row = jnp.full((L,), r, jnp.int32)
                    outb[r, pl.ds(g * L, L)] = plsc.load_gather(inb, [row, lo])
                    outb[r, pl.ds(HALF + g * L, L)] = plsc.load_gather(inb, [row, hi])

            pltpu.async_copy(outb, out_slice(c), osems[b])

            @pl.when(c + NIBUF < NCHUNK)
            def _():
                pltpu.async_copy(in_slice(c + NIBUF), inb, isems[b % NIBUF])

        return ()

    lax.fori_loop(0, NCHUNK // NOBUF, quad_body, ())
    for b in range(NOBUF):
        c_last = NCHUNK - NOBUF + b
        pltpu.make_async_copy(outs[b], out_slice(c_last), osems[b]).wait()


def kernel(x, perm, inv_perm):
    del inv_perm
    p = perm.astype(jnp.int32)
    packed = p[:HALF] | (p[HALF:] << 16)
    return _permute_cols(x, packed)


# speedup vs baseline: 1.0049x; 1.0049x over previous
import functools

import jax
import jax.numpy as jnp
from jax import lax
from jax.experimental import pallas as pl
from jax.experimental.pallas import tpu as pltpu
from jax.experimental.pallas import tpu_sc as plsc

NROWS = 16384
DIM = 4096
HALF = DIM // 2
NC, NS, L = 2, 16, 16
NW = NC * NS
ROWS_PER_W = NROWS // NW
R = 4
NCHUNK = ROWS_PER_W // R
NGROUP = HALF // L
NBUF = 2

_mesh = plsc.VectorSubcoreMesh(
    core_axis_name="c", subcore_axis_name="s", num_cores=NC, num_subcores=NS
)


@functools.partial(
    pl.kernel,
    out_type=jax.ShapeDtypeStruct((NROWS, DIM), jnp.float32),
    mesh=_mesh,
    compiler_params=pltpu.CompilerParams(needs_layout_passes=False),
    scratch_types=[
        pltpu.VMEM((HALF,), jnp.int32),
        pltpu.VMEM((R, DIM), jnp.float32),
        pltpu.VMEM((R, DIM), jnp.float32),
        pltpu.VMEM((R, DIM), jnp.float32),
        pltpu.VMEM((R, DIM), jnp.float32),
        pltpu.SemaphoreType.DMA,
        pltpu.SemaphoreType.DMA,
        pltpu.SemaphoreType.DMA,
        pltpu.SemaphoreType.DMA,
    ],
)
def _permute_cols(x_hbm, perm_hbm, out_hbm, idx_v, in0, in1, o0, o1,
                  is0, is1, os0, os1):
    ins, outs = (in0, in1), (o0, o1)
    isems, osems = (is0, is1), (os0, os1)
    wid = lax.axis_index("s") * NC + lax.axis_index("c")
    base = wid * ROWS_PER_W
    pltpu.sync_copy(perm_hbm, idx_v)

    def in_slice(c):
        return x_hbm.at[pl.ds(base + c * R, R)]

    def out_slice(c):
        return out_hbm.at[pl.ds(base + c * R, R)]

    pltpu.async_copy(in_slice(0), ins[0], isems[0])
    pltpu.async_copy(in_slice(1), ins[1], isems[1])

    def pair_body(p, _):
        for b in range(NBUF):
            c = NBUF * p + b
            inb, outb = ins[b], outs[b]
            pltpu.make_async_copy(in_slice(c), inb, isems[b]).wait()

            @pl.when(p >= 1)
            def _():
                pltpu.make_async_copy(outb, out_slice(c - NBUF), osems[b]).wait()

            @plsc.parallel_loop(0, NGROUP, 1, unroll=4)
            def _(g):
                packed = idx_v[pl.ds(g * L, L)]
                lo = jnp.bitwise_and(packed, jnp.full((L,), 0xFFFF, jnp.int32))
                hi = lax.shift_right_logical(packed, jnp.full((L,), 16, jnp.int32))
                for r in range(R):
                    row = jnp.full((L,), r, jnp.int32)
                    outb[r, pl.ds(g * L, L)] = plsc.load_gather(inb, [row, lo])
                    outb[r, pl.ds(HALF + g * L, L)] = plsc.load_gather(inb, [row, hi])

            pltpu.async_copy(outb, out_slice(c), osems[b])

            @pl.when(c + NBUF < NCHUNK)
            def _():
                pltpu.async_copy(in_slice(c + NBUF), inb, isems[b])

        return ()

    lax.fori_loop(0, NCHUNK // NBUF, pair_body, ())
    for b in range(NBUF):
        c_last = NCHUNK - NBUF + b
        pltpu.make_async_copy(outs[b], out_slice(c_last), osems[b]).wait()


def kernel(x, perm, inv_perm):
    del inv_perm
    p = perm.astype(jnp.int32)
    packed = p[:HALF] | (p[HALF:] << 16)
    return _permute_cols(x, packed)


# speedup vs baseline: 1.0066x; 1.0017x over previous
import functools

import jax
import jax.numpy as jnp
from jax import lax
from jax.experimental import pallas as pl
from jax.experimental.pallas import tpu as pltpu
from jax.experimental.pallas import tpu_sc as plsc

NROWS = 16384
DIM = 4096
HALF = DIM // 2
NC, NS, L = 2, 16, 16
NW = NC * NS
ROWS_PER_W = NROWS // NW
R = 4
NCHUNK = ROWS_PER_W // R
NGROUP = HALF // L
NBUF = 2

_mesh = plsc.VectorSubcoreMesh(
    core_axis_name="c", subcore_axis_name="s", num_cores=NC, num_subcores=NS
)


@functools.partial(
    pl.kernel,
    out_type=jax.ShapeDtypeStruct((NROWS, DIM), jnp.float32),
    mesh=_mesh,
    compiler_params=pltpu.CompilerParams(needs_layout_passes=False),
    scratch_types=[
        pltpu.VMEM((HALF,), jnp.int32),
        pltpu.VMEM((R, DIM), jnp.float32),
        pltpu.VMEM((R, DIM), jnp.float32),
        pltpu.VMEM((R, DIM), jnp.float32),
        pltpu.VMEM((R, DIM), jnp.float32),
        pltpu.SemaphoreType.DMA,
        pltpu.SemaphoreType.DMA,
        pltpu.SemaphoreType.DMA,
        pltpu.SemaphoreType.DMA,
    ],
)
def _permute_cols(x_hbm, perm_hbm, out_hbm, idx_v, in0, in1, o0, o1,
                  is0, is1, os0, os1):
    ins, outs = (in0, in1), (o0, o1)
    isems, osems = (is0, is1), (os0, os1)
    wid = lax.axis_index("s") * NC + lax.axis_index("c")
    base = wid * ROWS_PER_W

    def in_slice(c):
        return x_hbm.at[pl.ds(base + c * R, R)]

    def out_slice(c):
        return out_hbm.at[pl.ds(base + c * R, R)]

    pltpu.async_copy(in_slice(0), ins[0], isems[0])
    pltpu.async_copy(in_slice(1), ins[1], isems[1])
    pltpu.sync_copy(perm_hbm, idx_v)

    def pair_body(p, _):
        for b in range(NBUF):
            c = NBUF * p + b
            inb, outb = ins[b], outs[b]
            pltpu.make_async_copy(in_slice(c), inb, isems[b]).wait()

            @pl.when(p >= 1)
            def _():
                pltpu.make_async_copy(outb, out_slice(c - NBUF), osems[b]).wait()

            @plsc.parallel_loop(0, NGROUP, 1, unroll=4)
            def _(g):
                packed = idx_v[pl.ds(g * L, L)]
                lo = jnp.bitwise_and(packed, jnp.full((L,), 0xFFFF, jnp.int32))
                hi = lax.shift_right_logical(packed, jnp.full((L,), 16, jnp.int32))
                for r in range(R):
                    row = jnp.full((L,), r, jnp.int32)
                    outb[r, pl.ds(g * L, L)] = plsc.load_gather(inb, [row, lo])
                    outb[r, pl.ds(HALF + g * L, L)] = plsc.load_gather(inb, [row, hi])

            pltpu.async_copy(outb, out_slice(c), osems[b])

            @pl.when(c + NBUF < NCHUNK)
            def _():
                pltpu.async_copy(in_slice(c + NBUF), inb, isems[b])

        return ()

    lax.fori_loop(0, NCHUNK // NBUF, pair_body, ())
    for b in range(NBUF):
        c_last = NCHUNK - NBUF + b
        pltpu.make_async_copy(outs[b], out_slice(c_last), osems[b]).wait()


def kernel(x, perm, inv_perm):
    del inv_perm
    p = perm.astype(jnp.int32)
    packed = p[:HALF] | (p[HALF:] << 16)
    return _permute_cols(x, packed)
